# baseline (speedup 1.0000x reference)
import jax
import jax.numpy as jnp
from jax.experimental import pallas as pl
from jax.experimental.pallas import tpu as pltpu

HIDDEN = 1024
EXPERT_INTER = 1024
NUM_EXPERTS = 16
VOCAB = 100000
N_TOKENS = 128


def _moe_body(tid_ref, x_ref, gatew_ref, upw_ref, dnw_ref, out_ref):
    e = pl.program_id(0)

    @pl.when(e == 0)
    def _init():
        out_ref[...] = jnp.zeros_like(out_ref)

    tid = jnp.clip(tid_ref[...], 0, VOCAB - 1)
    eid = jax.lax.rem(tid, NUM_EXPERTS)
    mask = eid == e
    x = jnp.where(mask, x_ref[...], jnp.bfloat16(0.0))
    gate = jnp.dot(x, gatew_ref[0].astype(jnp.bfloat16),
                   preferred_element_type=jnp.float32)
    up = jnp.dot(x, upw_ref[0].astype(jnp.bfloat16),
                 preferred_element_type=jnp.float32)
    act = gate * jax.nn.sigmoid(gate) * up
    y = jnp.dot(act.astype(jnp.bfloat16), dnw_ref[0].astype(jnp.bfloat16),
                preferred_element_type=jnp.float32)
    out_ref[...] += y


def kernel(x, token_ids, gate_up_proj, down_proj):
    n = x.shape[0]
    x = x.astype(jnp.bfloat16)
    tid2d = token_ids.reshape(n, 1).astype(jnp.int32)
    return pl.pallas_call(
        _moe_body,
        grid=(NUM_EXPERTS,),
        in_specs=[
            pl.BlockSpec((n, 1), lambda e: (0, 0)),
            pl.BlockSpec((n, HIDDEN), lambda e: (0, 0)),
            pl.BlockSpec((1, HIDDEN, EXPERT_INTER), lambda e: (e, 0, 0)),
            pl.BlockSpec((1, HIDDEN, EXPERT_INTER), lambda e: (e, 0, 1)),
            pl.BlockSpec((1, EXPERT_INTER, HIDDEN), lambda e: (e, 0, 0)),
        ],
        out_specs=pl.BlockSpec((n, HIDDEN), lambda e: (0, 0)),
        out_shape=jax.ShapeDtypeStruct((n, HIDDEN), jnp.float32),
        compiler_params=pltpu.CompilerParams(
            dimension_semantics=("arbitrary",),
        ),
    )(tid2d, x, gate_up_proj, gate_up_proj, down_proj)


# speedup vs baseline: 1.0310x; 1.0310x over previous
import jax
import jax.numpy as jnp
from jax.experimental import pallas as pl
from jax.experimental.pallas import tpu as pltpu

HIDDEN = 1024
EXPERT_INTER = 1024
NUM_EXPERTS = 16
VOCAB = 100000
N_TOKENS = 128
NBUF = 2


def _moe_body(tid_ref, x_ref, gu_hbm, dn_hbm, out_ref,
              gateb, upb, dnb, gsem, usem, dsem):
    def gate_copy(e, slot):
        return pltpu.make_async_copy(
            gu_hbm.at[e, :, 0:EXPERT_INTER], gateb.at[slot], gsem.at[slot])

    def up_copy(e, slot):
        return pltpu.make_async_copy(
            gu_hbm.at[e, :, EXPERT_INTER:2 * EXPERT_INTER],
            upb.at[slot], usem.at[slot])

    def dn_copy(e, slot):
        return pltpu.make_async_copy(dn_hbm.at[e], dnb.at[slot], dsem.at[slot])

    def start_expert(e, slot):
        gate_copy(e, slot).start()
        up_copy(e, slot).start()
        dn_copy(e, slot).start()

    for p in range(NBUF):
        start_expert(p, p)

    tid = jnp.clip(tid_ref[...], 0, VOCAB - 1)
    eid = jax.lax.rem(tid, NUM_EXPERTS)

    acc = jnp.zeros((N_TOKENS, HIDDEN), jnp.float32)
    for e in range(NUM_EXPERTS):
        slot = e % NBUF
        mask = eid == e
        x = jnp.where(mask, x_ref[...], 0.0).astype(jnp.bfloat16)
        gate_copy(e, slot).wait()
        gate = jnp.dot(x, gateb[slot].astype(jnp.bfloat16),
                       preferred_element_type=jnp.float32)
        up_copy(e, slot).wait()
        up = jnp.dot(x, upb[slot].astype(jnp.bfloat16),
                     preferred_element_type=jnp.float32)
        act = (gate * jax.nn.sigmoid(gate) * up).astype(jnp.bfloat16)
        dn_copy(e, slot).wait()
        acc = acc + jnp.dot(act, dnb[slot].astype(jnp.bfloat16),
                            preferred_element_type=jnp.float32)
        if e + NBUF < NUM_EXPERTS:
            start_expert(e + NBUF, slot)
    out_ref[...] = acc


def kernel(x, token_ids, gate_up_proj, down_proj):
    n = x.shape[0]
    tid2d = token_ids.reshape(n, 1).astype(jnp.int32)
    return pl.pallas_call(
        _moe_body,
        in_specs=[
            pl.BlockSpec(memory_space=pltpu.MemorySpace.VMEM),
            pl.BlockSpec(memory_space=pltpu.MemorySpace.VMEM),
            pl.BlockSpec(memory_space=pltpu.MemorySpace.HBM),
            pl.BlockSpec(memory_space=pltpu.MemorySpace.HBM),
        ],
        out_specs=pl.BlockSpec(memory_space=pltpu.MemorySpace.VMEM),
        out_shape=jax.ShapeDtypeStruct((n, HIDDEN), jnp.float32),
        scratch_shapes=[
            pltpu.VMEM((NBUF, HIDDEN, EXPERT_INTER), jnp.float32),
            pltpu.VMEM((NBUF, HIDDEN, EXPERT_INTER), jnp.float32),
            pltpu.VMEM((NBUF, EXPERT_INTER, HIDDEN), jnp.float32),
            pltpu.SemaphoreType.DMA((NBUF,)),
            pltpu.SemaphoreType.DMA((NBUF,)),
            pltpu.SemaphoreType.DMA((NBUF,)),
        ],
    )(tid2d, x, gate_up_proj, down_proj)
